# baseline (device time: 59382 ns/iter reference)
import jax
import jax.numpy as jnp
from jax import lax
from jax.experimental import pallas as pl
from jax.experimental.pallas import tpu as pltpu

N_LOCAL_EXPERTS = 2
CAPACITY = 320


def kernel(x, assign, W1, W2):
    t, d = x.shape
    c = CAPACITY
    my_x = lax.axis_index("x")
    x_bf = x.astype(jnp.bfloat16)

    e_mine = N_LOCAL_EXPERTS * my_x
    e_theirs = N_LOCAL_EXPERTS * (1 - my_x)

    ks = jnp.stack([e_mine, e_mine + 1, e_theirs, e_theirs + 1])
    m = assign[None, :] == ks[:, None]
    pos = jnp.where(m, jnp.cumsum(m.astype(jnp.int32), axis=1) - 1, -1)
    pos_col = pos[:, :, None]
    pos_row = pos[:, None, :]

    def body(x_ref, pc_ref, pr_ref, w1_hbm, w2_hbm, out_ref,
             xo_ref, xr_ref, rs_ref, rr_ref, w1_ref, w2_ref,
             wsems, send_sems, recv_sems):
        partner = (1 - lax.axis_index("x"), lax.axis_index("y"),
                   lax.axis_index("z"))

        wcp = []
        for j in range(N_LOCAL_EXPERTS):
            c1 = pltpu.make_async_copy(
                w1_hbm.at[j], w1_ref.at[j], wsems.at[2 * j])
            c2 = pltpu.make_async_copy(
                w2_hbm.at[j], w2_ref.at[j], wsems.at[2 * j + 1])
            c1.start()
            c2.start()
            wcp.append((c1, c2))

        def qt(j):
            return (lax.broadcasted_iota(jnp.int32, (c, t), 0)
                    == pr_ref[j]).astype(jnp.bfloat16)

        def q(j):
            return (pc_ref[j]
                    == lax.broadcasted_iota(jnp.int32, (t, c), 1)
                    ).astype(jnp.bfloat16)

        def gather(j):
            return jnp.dot(qt(j), x_ref[:, :],
                           preferred_element_type=jnp.float32)

        def ffn(xg, j):
            h = jnp.maximum(
                jnp.dot(xg, w1_ref[j], preferred_element_type=jnp.float32),
                0.0)
            return jnp.dot(h, w2_ref[j], preferred_element_type=jnp.float32)

        xo_ref[0, :, :] = gather(2).astype(jnp.bfloat16)
        xo_ref[1, :, :] = gather(3).astype(jnp.bfloat16)

        barrier = pltpu.get_barrier_semaphore()
        pl.semaphore_signal(barrier, inc=1, device_id=partner,
                            device_id_type=pl.DeviceIdType.MESH)
        pl.semaphore_wait(barrier, 1)

        send_x = pltpu.make_async_remote_copy(
            src_ref=xo_ref, dst_ref=xr_ref,
            send_sem=send_sems.at[0], recv_sem=recv_sems.at[0],
            device_id=partner, device_id_type=pl.DeviceIdType.MESH)
        send_x.start()

        xg0 = gather(0)
        wcp[0][0].wait()
        wcp[0][1].wait()
        p0 = ffn(xg0, 0).astype(jnp.bfloat16)
        xg1 = gather(1)
        wcp[1][0].wait()
        wcp[1][1].wait()
        p1 = ffn(xg1, 1).astype(jnp.bfloat16)
        out_ref[:, :] = (
            jnp.dot(q(0), p0, preferred_element_type=jnp.float32)
            + jnp.dot(q(1), p1, preferred_element_type=jnp.float32))

        send_x.wait()

        rs_ref[0, :, :] = ffn(
            xr_ref[0].astype(jnp.float32), 0).astype(jnp.bfloat16)
        ret0 = pltpu.make_async_remote_copy(
            src_ref=rs_ref.at[0], dst_ref=rr_ref.at[0],
            send_sem=send_sems.at[1], recv_sem=recv_sems.at[1],
            device_id=partner, device_id_type=pl.DeviceIdType.MESH)
        ret0.start()

        hh = c // 2
        rs_ref[1, :hh, :] = ffn(
            xr_ref[1, :hh].astype(jnp.float32), 1).astype(jnp.bfloat16)
        ret1a = pltpu.make_async_remote_copy(
            src_ref=rs_ref.at[1, pl.ds(0, hh)],
            dst_ref=rr_ref.at[1, pl.ds(0, hh)],
            send_sem=send_sems.at[2], recv_sem=recv_sems.at[2],
            device_id=partner, device_id_type=pl.DeviceIdType.MESH)
        ret1a.start()

        rs_ref[1, hh:, :] = ffn(
            xr_ref[1, hh:].astype(jnp.float32), 1).astype(jnp.bfloat16)
        ret1b = pltpu.make_async_remote_copy(
            src_ref=rs_ref.at[1, pl.ds(hh, hh)],
            dst_ref=rr_ref.at[1, pl.ds(hh, hh)],
            send_sem=send_sems.at[3], recv_sem=recv_sems.at[3],
            device_id=partner, device_id_type=pl.DeviceIdType.MESH)
        ret1b.start()

        ret0.wait()
        s0 = jnp.dot(q(2), rr_ref[0, :, :],
                     preferred_element_type=jnp.float32)
        q3 = q(3)
        ret1a.wait()
        s1a = jnp.dot(q3[:, :hh], rr_ref[1, :hh, :],
                      preferred_element_type=jnp.float32)
        ret1b.wait()
        out_ref[:, :] = out_ref[:, :] + s0 + s1a + jnp.dot(
            q3[:, hh:], rr_ref[1, hh:, :],
            preferred_element_type=jnp.float32)

    return pl.pallas_call(
        body,
        out_shape=jax.ShapeDtypeStruct((t, d), jnp.float32),
        in_specs=[
            pl.BlockSpec(memory_space=pltpu.VMEM),
            pl.BlockSpec(memory_space=pltpu.VMEM),
            pl.BlockSpec(memory_space=pltpu.VMEM),
            pl.BlockSpec(memory_space=pltpu.MemorySpace.HBM),
            pl.BlockSpec(memory_space=pltpu.MemorySpace.HBM),
        ],
        out_specs=pl.BlockSpec(memory_space=pltpu.VMEM),
        scratch_shapes=[
            pltpu.VMEM((2, c, d), jnp.bfloat16),
            pltpu.VMEM((2, c, d), jnp.bfloat16),
            pltpu.VMEM((2, c, d), jnp.bfloat16),
            pltpu.VMEM((2, c, d), jnp.bfloat16),
            pltpu.VMEM(W1.shape, jnp.float32),
            pltpu.VMEM(W2.shape, jnp.float32),
            pltpu.SemaphoreType.DMA((4,)),
            pltpu.SemaphoreType.DMA((4,)),
            pltpu.SemaphoreType.DMA((4,)),
        ],
        compiler_params=pltpu.CompilerParams(
            collective_id=0, vmem_limit_bytes=100 * 1024 * 1024),
    )(x_bf, pos_col, pos_row, W1, W2)


# device time: 57638 ns/iter; 1.0303x vs baseline; 1.0303x over previous
import jax
import jax.numpy as jnp
from jax import lax
from jax.experimental import pallas as pl
from jax.experimental.pallas import tpu as pltpu

N_LOCAL_EXPERTS = 2
CAPACITY = 320


def kernel(x, assign, W1, W2):
    t, d = x.shape
    c = CAPACITY
    my_x = lax.axis_index("x")
    x_bf = x.astype(jnp.bfloat16)

    e_mine = N_LOCAL_EXPERTS * my_x
    e_theirs = N_LOCAL_EXPERTS * (1 - my_x)

    ks = jnp.stack([e_mine, e_mine + 1, e_theirs, e_theirs + 1])
    m = assign[None, :] == ks[:, None]
    pos = jnp.where(m, jnp.cumsum(m.astype(jnp.int32), axis=1) - 1, -1)
    pos_col = pos[:, :, None]
    pos_row = pos[:, None, :]

    def body(x_ref, pc_ref, pr_ref, w1_hbm, w2_hbm, out_ref,
             xo_ref, xr_ref, rs_ref, rr_ref, w1_ref, w2_ref,
             wsems, send_sems, recv_sems):
        partner = (1 - lax.axis_index("x"), lax.axis_index("y"),
                   lax.axis_index("z"))

        wcp = []
        for j in range(N_LOCAL_EXPERTS):
            c1 = pltpu.make_async_copy(
                w1_hbm.at[j], w1_ref.at[j], wsems.at[2 * j])
            c2 = pltpu.make_async_copy(
                w2_hbm.at[j], w2_ref.at[j], wsems.at[2 * j + 1])
            c1.start()
            c2.start()
            wcp.append((c1, c2))

        def qt(j):
            return (lax.broadcasted_iota(jnp.int32, (c, t), 0)
                    == pr_ref[j]).astype(jnp.bfloat16)

        def q(j):
            return (pc_ref[j]
                    == lax.broadcasted_iota(jnp.int32, (t, c), 1)
                    ).astype(jnp.bfloat16)

        def gather(j):
            return jnp.dot(qt(j), x_ref[:, :],
                           preferred_element_type=jnp.float32)

        def ffn(xg, j):
            h = jnp.maximum(
                jnp.dot(xg, w1_ref[j], preferred_element_type=jnp.float32),
                0.0)
            return jnp.dot(h, w2_ref[j], preferred_element_type=jnp.float32)

        xo_ref[0, :, :] = gather(2).astype(jnp.bfloat16)
        xo_ref[1, :, :] = gather(3).astype(jnp.bfloat16)

        barrier = pltpu.get_barrier_semaphore()
        pl.semaphore_signal(barrier, inc=1, device_id=partner,
                            device_id_type=pl.DeviceIdType.MESH)
        pl.semaphore_wait(barrier, 1)

        send_x = pltpu.make_async_remote_copy(
            src_ref=xo_ref, dst_ref=xr_ref,
            send_sem=send_sems.at[0], recv_sem=recv_sems.at[0],
            device_id=partner, device_id_type=pl.DeviceIdType.MESH)
        send_x.start()

        xg0 = gather(0)
        wcp[0][0].wait()
        wcp[0][1].wait()
        p0 = ffn(xg0, 0).astype(jnp.bfloat16)
        xg1 = gather(1)
        wcp[1][0].wait()
        wcp[1][1].wait()
        p1 = ffn(xg1, 1).astype(jnp.bfloat16)

        send_x.wait()

        rs_ref[0, :, :] = ffn(
            xr_ref[0].astype(jnp.float32), 0).astype(jnp.bfloat16)
        ret0 = pltpu.make_async_remote_copy(
            src_ref=rs_ref.at[0], dst_ref=rr_ref.at[0],
            send_sem=send_sems.at[1], recv_sem=recv_sems.at[1],
            device_id=partner, device_id_type=pl.DeviceIdType.MESH)
        ret0.start()

        hh = c // 2
        rs_ref[1, :hh, :] = ffn(
            xr_ref[1, :hh].astype(jnp.float32), 1).astype(jnp.bfloat16)
        ret1a = pltpu.make_async_remote_copy(
            src_ref=rs_ref.at[1, pl.ds(0, hh)],
            dst_ref=rr_ref.at[1, pl.ds(0, hh)],
            send_sem=send_sems.at[2], recv_sem=recv_sems.at[2],
            device_id=partner, device_id_type=pl.DeviceIdType.MESH)
        ret1a.start()

        rs_ref[1, hh:, :] = ffn(
            xr_ref[1, hh:].astype(jnp.float32), 1).astype(jnp.bfloat16)
        ret1b = pltpu.make_async_remote_copy(
            src_ref=rs_ref.at[1, pl.ds(hh, hh)],
            dst_ref=rr_ref.at[1, pl.ds(hh, hh)],
            send_sem=send_sems.at[3], recv_sem=recv_sems.at[3],
            device_id=partner, device_id_type=pl.DeviceIdType.MESH)
        ret1b.start()

        out_ref[:, :] = (
            jnp.dot(q(0), p0, preferred_element_type=jnp.float32)
            + jnp.dot(q(1), p1, preferred_element_type=jnp.float32))

        ret0.wait()
        s0 = jnp.dot(q(2), rr_ref[0, :, :],
                     preferred_element_type=jnp.float32)
        q3 = q(3)
        ret1a.wait()
        s1a = jnp.dot(q3[:, :hh], rr_ref[1, :hh, :],
                      preferred_element_type=jnp.float32)
        ret1b.wait()
        out_ref[:, :] = out_ref[:, :] + s0 + s1a + jnp.dot(
            q3[:, hh:], rr_ref[1, hh:, :],
            preferred_element_type=jnp.float32)

    return pl.pallas_call(
        body,
        out_shape=jax.ShapeDtypeStruct((t, d), jnp.float32),
        in_specs=[
            pl.BlockSpec(memory_space=pltpu.VMEM),
            pl.BlockSpec(memory_space=pltpu.VMEM),
            pl.BlockSpec(memory_space=pltpu.VMEM),
            pl.BlockSpec(memory_space=pltpu.MemorySpace.HBM),
            pl.BlockSpec(memory_space=pltpu.MemorySpace.HBM),
        ],
        out_specs=pl.BlockSpec(memory_space=pltpu.VMEM),
        scratch_shapes=[
            pltpu.VMEM((2, c, d), jnp.bfloat16),
            pltpu.VMEM((2, c, d), jnp.bfloat16),
            pltpu.VMEM((2, c, d), jnp.bfloat16),
            pltpu.VMEM((2, c, d), jnp.bfloat16),
            pltpu.VMEM(W1.shape, jnp.float32),
            pltpu.VMEM(W2.shape, jnp.float32),
            pltpu.SemaphoreType.DMA((4,)),
            pltpu.SemaphoreType.DMA((4,)),
            pltpu.SemaphoreType.DMA((4,)),
        ],
        compiler_params=pltpu.CompilerParams(
            collective_id=0, vmem_limit_bytes=100 * 1024 * 1024),
    )(x_bf, pos_col, pos_row, W1, W2)
